# baseline (speedup 1.0000x reference)
import jax
import jax.numpy as jnp
from jax import lax
from jax.experimental import pallas as pl
from jax.experimental.pallas import tpu as pltpu
from jax.experimental.pallas import tpu_sc as plsc

_L = 128
_H = 8
_U = 25
_NL = 16
_NC = _L // _NL

_NEG = -3.0e38
_POS = 3.0e38

_IDX_K = (11, 16, 17, 21, 23, 26, 28, 30, 53, 55, 60, 69, 70, 77,
          85, 91, 96, 100, 103, 104, 109, 110, 114, 116)
_DUP_IDX = 60


def _sc_body(q_hbm, k_hbm, v_hbm, out_hbm, qv, kv, vv, ov):
    cid = lax.axis_index("c")
    sid = lax.axis_index("s")
    h = cid * 4 + sid

    @pl.when(sid < 4)
    def _work():
        pltpu.sync_copy(q_hbm.at[h], qv)
        pltpu.sync_copy(k_hbm.at[h], kv)
        pltpu.sync_copy(v_hbm.at[h], vv)

        iota = lax.broadcasted_iota(jnp.int32, (_NL,), 0)

        ssum = jnp.float32(0.0)
        smax = jnp.float32(_NEG)
        smin = jnp.float32(_POS)
        for c in range(_NC):
            in_chunk = [i - c * _NL for i in _IDX_K
                        if c * _NL <= i < (c + 1) * _NL]
            if not in_chunk:
                continue
            kc = kv[pl.ds(c * _NL, _NL)]
            mem = iota == in_chunk[0]
            for loc in in_chunk[1:]:
                mem = mem | (iota == loc)
            cnt = mem.astype(jnp.float32)
            if c * _NL <= _DUP_IDX < (c + 1) * _NL:
                cnt = cnt + (iota == (_DUP_IDX - c * _NL)).astype(jnp.float32)
            ssum = ssum + jnp.sum(kc * cnt)
            smax = jnp.maximum(smax, jnp.max(jnp.where(mem, kc, _NEG)))
            smin = jnp.minimum(smin, jnp.min(jnp.where(mem, kc, _POS)))
        a = smax - ssum * (1.0 / _L)
        b = smin - ssum * (1.0 / _L)

        ms = []
        for c in range(_NC):
            qc = qv[pl.ds(c * _NL, _NL)]
            ms.append(jnp.where(qc >= 0.0, qc * a, qc * b))

        thi = jnp.full((_NL,), _NEG, jnp.float32)
        tlo = jnp.full((_NL,), _NEG, jnp.float32)
        for c in range(_NC):
            s = jnp.sort(ms[c])
            rs = jnp.flip(s, 0)
            hi2 = jnp.sort(jnp.maximum(thi, rs))
            lo1 = jnp.sort(jnp.minimum(thi, rs))
            tlo = jnp.sort(jnp.maximum(tlo, jnp.flip(lo1, 0)))
            thi = hi2
        thr = jnp.max(jnp.where(iota == 7, tlo, _NEG))

        ngt = jnp.float32(0.0)
        for c in range(_NC):
            ngt = ngt + jnp.sum((ms[c] > thr).astype(jnp.float32))
        quota = jnp.float32(_U) - ngt

        carry = jnp.float32(0.0)
        eqoff = jnp.float32(0.0)
        for c in range(_NC):
            vc = vv[pl.ds(c * _NL, _NL)]
            pc = jnp.cumsum(vc) + carry
            carry = carry + jnp.sum(vc)
            n = (iota + (c * _NL + 1)).astype(jnp.float32)
            eqf = (ms[c] == thr).astype(jnp.float32)
            incl = jnp.cumsum(eqf)
            ex = incl - eqf + eqoff
            eqoff = eqoff + jnp.max(incl)
            sel = (ms[c] > thr) | ((eqf > 0.0) & (ex < quota))
            ov[pl.ds(c * _NL, _NL)] = jnp.where(sel, pc / n, vc)

        pltpu.sync_copy(ov, out_hbm.at[h])


def kernel(queries, keys, values):
    q = queries.reshape(_H, _L)
    k = keys.reshape(_H, _L)
    v = values.reshape(_H, _L)

    mesh = plsc.VectorSubcoreMesh(core_axis_name="c", subcore_axis_name="s")
    run = pl.kernel(
        _sc_body,
        out_type=jax.ShapeDtypeStruct((_H, _L), jnp.float32),
        mesh=mesh,
        compiler_params=pltpu.CompilerParams(needs_layout_passes=False),
        scratch_types=[
            pltpu.VMEM((_L,), jnp.float32),
            pltpu.VMEM((_L,), jnp.float32),
            pltpu.VMEM((_L,), jnp.float32),
            pltpu.VMEM((_L,), jnp.float32),
        ],
    )
    out = run(q, k, v)
    return out.reshape(1, _H, _L, 1)


# speedup vs baseline: 1.0390x; 1.0390x over previous
import jax
import jax.numpy as jnp
from jax import lax
from jax.experimental import pallas as pl
from jax.experimental.pallas import tpu as pltpu
from jax.experimental.pallas import tpu_sc as plsc

_L = 128
_H = 8
_U = 25
_NL = 16
_NC = _L // _NL

_NEG = -3.0e38
_POS = 3.0e38

_IDX_K = (11, 16, 17, 21, 23, 26, 28, 30, 53, 55, 60, 69, 70, 77,
          85, 91, 96, 100, 103, 104, 109, 110, 114, 116)
_DUP_IDX = 60


def _sc_body(q_hbm, k_hbm, v_hbm, out_hbm, qv, kv, vv, ov, sem):
    cid = lax.axis_index("c")
    sid = lax.axis_index("s")
    h = cid * 4 + sid

    @pl.when(sid < 4)
    def _work():
        cq = pltpu.async_copy(q_hbm.at[h], qv, sem)
        ck = pltpu.async_copy(k_hbm.at[h], kv, sem)
        cv = pltpu.async_copy(v_hbm.at[h], vv, sem)
        ck.wait()
        cq.wait()
        cv.wait()

        iota = lax.broadcasted_iota(jnp.int32, (_NL,), 0)

        ssum = jnp.float32(0.0)
        smax = jnp.float32(_NEG)
        smin = jnp.float32(_POS)
        for c in range(_NC):
            in_chunk = [i - c * _NL for i in _IDX_K
                        if c * _NL <= i < (c + 1) * _NL]
            if not in_chunk:
                continue
            kc = kv[pl.ds(c * _NL, _NL)]
            mem = iota == in_chunk[0]
            for loc in in_chunk[1:]:
                mem = mem | (iota == loc)
            cnt = mem.astype(jnp.float32)
            if c * _NL <= _DUP_IDX < (c + 1) * _NL:
                cnt = cnt + (iota == (_DUP_IDX - c * _NL)).astype(jnp.float32)
            ssum = ssum + jnp.sum(kc * cnt)
            smax = jnp.maximum(smax, jnp.max(jnp.where(mem, kc, _NEG)))
            smin = jnp.minimum(smin, jnp.min(jnp.where(mem, kc, _POS)))
        a = smax - ssum * (1.0 / _L)
        b = smin - ssum * (1.0 / _L)

        ms = []
        for c in range(_NC):
            qc = qv[pl.ds(c * _NL, _NL)]
            ms.append(jnp.where(qc >= 0.0, qc * a, qc * b))

        thi = jnp.full((_NL,), _NEG, jnp.float32)
        tlo = jnp.full((_NL,), _NEG, jnp.float32)
        for c in range(_NC):
            s = jnp.sort(ms[c])
            rs = jnp.flip(s, 0)
            hi2 = jnp.sort(jnp.maximum(thi, rs))
            lo1 = jnp.sort(jnp.minimum(thi, rs))
            tlo = jnp.sort(jnp.maximum(tlo, jnp.flip(lo1, 0)))
            thi = hi2
        thr = jnp.max(jnp.where(iota == 7, tlo, _NEG))

        ngt = jnp.float32(0.0)
        for c in range(_NC):
            ngt = ngt + jnp.sum((ms[c] > thr).astype(jnp.float32))
        quota = jnp.float32(_U) - ngt

        carry = jnp.float32(0.0)
        eqoff = jnp.float32(0.0)
        for c in range(_NC):
            vc = vv[pl.ds(c * _NL, _NL)]
            pc = jnp.cumsum(vc) + carry
            carry = carry + jnp.sum(vc)
            n = (iota + (c * _NL + 1)).astype(jnp.float32)
            eqf = (ms[c] == thr).astype(jnp.float32)
            incl = jnp.cumsum(eqf)
            ex = incl - eqf + eqoff
            eqoff = eqoff + jnp.max(incl)
            sel = (ms[c] > thr) | ((eqf > 0.0) & (ex < quota))
            ov[pl.ds(c * _NL, _NL)] = jnp.where(sel, pc / n, vc)

        pltpu.sync_copy(ov, out_hbm.at[h])


def kernel(queries, keys, values):
    q = queries.reshape(_H, _L)
    k = keys.reshape(_H, _L)
    v = values.reshape(_H, _L)

    mesh = plsc.VectorSubcoreMesh(core_axis_name="c", subcore_axis_name="s")
    run = pl.kernel(
        _sc_body,
        out_type=jax.ShapeDtypeStruct((_H, _L), jnp.float32),
        mesh=mesh,
        compiler_params=pltpu.CompilerParams(needs_layout_passes=False),
        scratch_types=[
            pltpu.VMEM((_L,), jnp.float32),
            pltpu.VMEM((_L,), jnp.float32),
            pltpu.VMEM((_L,), jnp.float32),
            pltpu.VMEM((_L,), jnp.float32),
            pltpu.SemaphoreType.DMA,
        ],
    )
    out = run(q, k, v)
    return out.reshape(1, _H, _L, 1)


# speedup vs baseline: 1.1186x; 1.0765x over previous
import jax
import jax.numpy as jnp
from jax import lax
from jax.experimental import pallas as pl
from jax.experimental.pallas import tpu as pltpu
from jax.experimental.pallas import tpu_sc as plsc

_L = 128
_H = 8
_U = 25
_NL = 16
_NC = _L // _NL

_NEG = -3.0e38
_POS = 3.0e38

_IDX_K = (11, 16, 17, 21, 23, 26, 28, 30, 53, 55, 60, 69, 70, 77,
          85, 91, 96, 100, 103, 104, 109, 110, 114, 116)
_DUP_IDX = 60


def _sc_body(q_hbm, k_hbm, v_hbm, out_hbm, qv, kv, vv, ov, sem):
    cid = lax.axis_index("c")
    sid = lax.axis_index("s")
    h = cid * 8 + sid

    @pl.when(sid < 8)
    def _work():
        cq = pltpu.async_copy(q_hbm.at[h], qv, sem)
        ck = pltpu.async_copy(k_hbm.at[h], kv, sem)
        cv = pltpu.async_copy(v_hbm.at[h], vv, sem)
        ck.wait()
        cq.wait()
        cv.wait()

        iota = lax.broadcasted_iota(jnp.int32, (_NL,), 0)

        ssum = jnp.float32(0.0)
        smax = jnp.float32(_NEG)
        smin = jnp.float32(_POS)
        for c in range(_NC):
            in_chunk = [i - c * _NL for i in _IDX_K
                        if c * _NL <= i < (c + 1) * _NL]
            if not in_chunk:
                continue
            kc = kv[pl.ds(c * _NL, _NL)]
            mem = iota == in_chunk[0]
            for loc in in_chunk[1:]:
                mem = mem | (iota == loc)
            cnt = mem.astype(jnp.float32)
            if c * _NL <= _DUP_IDX < (c + 1) * _NL:
                cnt = cnt + (iota == (_DUP_IDX - c * _NL)).astype(jnp.float32)
            ssum = ssum + jnp.sum(kc * cnt)
            smax = jnp.maximum(smax, jnp.max(jnp.where(mem, kc, _NEG)))
            smin = jnp.minimum(smin, jnp.min(jnp.where(mem, kc, _POS)))
        a = smax - ssum * (1.0 / _L)
        b = smin - ssum * (1.0 / _L)

        ms = []
        for c in range(_NC):
            qc = qv[pl.ds(c * _NL, _NL)]
            ms.append(jnp.where(qc >= 0.0, qc * a, qc * b))

        thi = jnp.full((_NL,), _NEG, jnp.float32)
        tlo = jnp.full((_NL,), _NEG, jnp.float32)
        for c in range(_NC):
            s = jnp.sort(ms[c])
            rs = jnp.flip(s, 0)
            hi2 = jnp.sort(jnp.maximum(thi, rs))
            lo1 = jnp.sort(jnp.minimum(thi, rs))
            tlo = jnp.sort(jnp.maximum(tlo, jnp.flip(lo1, 0)))
            thi = hi2
        thr = jnp.max(jnp.where(iota == 7, tlo, _NEG))

        ngt = jnp.float32(0.0)
        for c in range(_NC):
            ngt = ngt + jnp.sum((ms[c] > thr).astype(jnp.float32))
        quota = jnp.float32(_U) - ngt

        carry = jnp.float32(0.0)
        eqoff = jnp.float32(0.0)
        for c in range(_NC):
            vc = vv[pl.ds(c * _NL, _NL)]
            pc = jnp.cumsum(vc) + carry
            carry = carry + jnp.sum(vc)
            n = (iota + (c * _NL + 1)).astype(jnp.float32)
            eqf = (ms[c] == thr).astype(jnp.float32)
            incl = jnp.cumsum(eqf)
            ex = incl - eqf + eqoff
            eqoff = eqoff + jnp.max(incl)
            sel = (ms[c] > thr) | ((eqf > 0.0) & (ex < quota))
            ov[pl.ds(c * _NL, _NL)] = jnp.where(sel, pc / n, vc)

        pltpu.sync_copy(ov, out_hbm.at[h])


def kernel(queries, keys, values):
    q = queries.reshape(_H, _L)
    k = keys.reshape(_H, _L)
    v = values.reshape(_H, _L)

    mesh = plsc.VectorSubcoreMesh(core_axis_name="c", subcore_axis_name="s",
                                  num_cores=1)
    run = pl.kernel(
        _sc_body,
        out_type=jax.ShapeDtypeStruct((_H, _L), jnp.float32),
        mesh=mesh,
        compiler_params=pltpu.CompilerParams(needs_layout_passes=False),
        scratch_types=[
            pltpu.VMEM((_L,), jnp.float32),
            pltpu.VMEM((_L,), jnp.float32),
            pltpu.VMEM((_L,), jnp.float32),
            pltpu.VMEM((_L,), jnp.float32),
            pltpu.SemaphoreType.DMA,
        ],
    )
    out = run(q, k, v)
    return out.reshape(1, _H, _L, 1)


# speedup vs baseline: 1.1215x; 1.0026x over previous
import jax
import jax.numpy as jnp
from jax import lax
from jax.experimental import pallas as pl
from jax.experimental.pallas import tpu as pltpu
from jax.experimental.pallas import tpu_sc as plsc

_L = 128
_H = 8
_U = 25
_NL = 16
_NC = _L // _NL

_NEG = -3.0e38
_POS = 3.0e38

_IDX_K = (11, 16, 17, 21, 23, 26, 28, 30, 53, 55, 60, 69, 70, 77,
          85, 91, 96, 100, 103, 104, 109, 110, 114, 116)
_DUP_IDX = 60


def _sc_body(q_hbm, k_hbm, v_hbm, out_hbm, qv, kv, vv, ov, sem):
    cid = lax.axis_index("c")
    sid = lax.axis_index("s")
    h = cid * 8 + sid

    @pl.when(sid < 8)
    def _work():
        cq = pltpu.async_copy(q_hbm.at[h], qv, sem)
        ck = pltpu.async_copy(k_hbm.at[h], kv, sem)
        cv = pltpu.async_copy(v_hbm.at[h], vv, sem)
        ck.wait()
        cq.wait()
        cv.wait()

        iota = lax.broadcasted_iota(jnp.int32, (_NL,), 0)

        ssum = jnp.float32(0.0)
        smax = jnp.float32(_NEG)
        smin = jnp.float32(_POS)
        for c in range(_NC):
            in_chunk = [i - c * _NL for i in _IDX_K
                        if c * _NL <= i < (c + 1) * _NL]
            if not in_chunk:
                continue
            kc = kv[pl.ds(c * _NL, _NL)]
            mem = iota == in_chunk[0]
            for loc in in_chunk[1:]:
                mem = mem | (iota == loc)
            cnt = mem.astype(jnp.float32)
            if c * _NL <= _DUP_IDX < (c + 1) * _NL:
                cnt = cnt + (iota == (_DUP_IDX - c * _NL)).astype(jnp.float32)
            ssum = ssum + jnp.sum(kc * cnt)
            smax = jnp.maximum(smax, jnp.max(jnp.where(mem, kc, _NEG)))
            smin = jnp.minimum(smin, jnp.min(jnp.where(mem, kc, _POS)))
        a = smax - ssum * (1.0 / _L)
        b = smin - ssum * (1.0 / _L)

        ms = []
        for c in range(_NC):
            qc = qv[pl.ds(c * _NL, _NL)]
            ms.append(jnp.where(qc >= 0.0, qc * a, qc * b))

        thi = jnp.full((_NL,), _NEG, jnp.float32)
        tlo = jnp.full((_NL,), _NEG, jnp.float32)
        for c in range(_NC):
            s = jnp.sort(ms[c])
            rs = jnp.flip(s, 0)
            hi2 = jnp.sort(jnp.maximum(thi, rs))
            lo1 = jnp.sort(jnp.minimum(thi, rs))
            tlo = jnp.sort(jnp.maximum(tlo, jnp.flip(lo1, 0)))
            thi = hi2
        thr = jnp.max(jnp.where(iota == 7, tlo, _NEG))

        ngt = jnp.float32(0.0)
        for c in range(_NC):
            ngt = ngt + jnp.sum((ms[c] > thr).astype(jnp.float32))
        quota = jnp.float32(_U) - ngt

        carry = jnp.float32(0.0)
        eqoff = jnp.float32(0.0)
        for c in range(_NC):
            vc = vv[pl.ds(c * _NL, _NL)]
            pc = jnp.cumsum(vc) + carry
            carry = carry + jnp.sum(vc)
            n = (iota + (c * _NL + 1)).astype(jnp.float32)
            eqf = (ms[c] == thr).astype(jnp.float32)
            incl = jnp.cumsum(eqf)
            ex = incl - eqf + eqoff
            eqoff = eqoff + jnp.max(incl)
            sel = (ms[c] > thr) | ((eqf > 0.0) & (ex < quota))
            ov[pl.ds(c * _NL, _NL)] = jnp.where(sel, pc / n, vc)

        pltpu.sync_copy(ov, out_hbm.at[h])


def kernel(queries, keys, values):
    q = queries.reshape(_H, _L)
    k = keys.reshape(_H, _L)
    v = values.reshape(_H, _L)

    mesh = plsc.VectorSubcoreMesh(core_axis_name="c", subcore_axis_name="s",
                                  num_cores=1, num_subcores=8)
    run = pl.kernel(
        _sc_body,
        out_type=jax.ShapeDtypeStruct((_H, _L), jnp.float32),
        mesh=mesh,
        compiler_params=pltpu.CompilerParams(needs_layout_passes=False),
        scratch_types=[
            pltpu.VMEM((_L,), jnp.float32),
            pltpu.VMEM((_L,), jnp.float32),
            pltpu.VMEM((_L,), jnp.float32),
            pltpu.VMEM((_L,), jnp.float32),
            pltpu.SemaphoreType.DMA,
        ],
    )
    out = run(q, k, v)
    return out.reshape(1, _H, _L, 1)


# speedup vs baseline: 1.1249x; 1.0030x over previous
import jax
import jax.numpy as jnp
from jax import lax
from jax.experimental import pallas as pl
from jax.experimental.pallas import tpu as pltpu
from jax.experimental.pallas import tpu_sc as plsc

_L = 128
_H = 8
_U = 25
_NL = 16
_NC = _L // _NL

_NEG = -3.0e38
_POS = 3.0e38

_IDX_K = (11, 16, 17, 21, 23, 26, 28, 30, 53, 55, 60, 69, 70, 77,
          85, 91, 96, 100, 103, 104, 109, 110, 114, 116)
_DUP_IDX = 60


def _sc_body(q_hbm, k_hbm, v_hbm, out_hbm, qv, kv, vv, ov, sem):
    cid = lax.axis_index("c")
    sid = lax.axis_index("s")
    h = cid * 8 + sid

    @pl.when(sid < 8)
    def _work():
        cq = pltpu.async_copy(q_hbm.at[h], qv, sem)
        ck = pltpu.async_copy(k_hbm.at[h], kv, sem)
        cv = pltpu.async_copy(v_hbm.at[h], vv, sem)
        ck.wait()
        cq.wait()
        cv.wait()

        iota = lax.broadcasted_iota(jnp.int32, (_NL,), 0)

        ssum = jnp.float32(0.0)
        smax = jnp.float32(_NEG)
        smin = jnp.float32(_POS)
        for c in range(_NC):
            in_chunk = [i - c * _NL for i in _IDX_K
                        if c * _NL <= i < (c + 1) * _NL]
            if not in_chunk:
                continue
            kc = kv[pl.ds(c * _NL, _NL)]
            mem = iota == in_chunk[0]
            for loc in in_chunk[1:]:
                mem = mem | (iota == loc)
            cnt = mem.astype(jnp.float32)
            if c * _NL <= _DUP_IDX < (c + 1) * _NL:
                cnt = cnt + (iota == (_DUP_IDX - c * _NL)).astype(jnp.float32)
            ssum = ssum + jnp.sum(kc * cnt)
            smax = jnp.maximum(smax, jnp.max(jnp.where(mem, kc, _NEG)))
            smin = jnp.minimum(smin, jnp.min(jnp.where(mem, kc, _POS)))
        a = smax - ssum * (1.0 / _L)
        b = smin - ssum * (1.0 / _L)

        ms = []
        for c in range(_NC):
            qc = qv[pl.ds(c * _NL, _NL)]
            ms.append(jnp.where(qc >= 0.0, qc * a, qc * b))

        thi = jnp.full((_NL,), _NEG, jnp.float32)
        tlo = jnp.full((_NL,), _NEG, jnp.float32)
        for c in range(_NC):
            s = jnp.sort(ms[c])
            rs = jnp.flip(s, 0)
            hi2 = jnp.sort(jnp.maximum(thi, rs))
            lo1 = jnp.sort(jnp.minimum(thi, rs))
            tlo = jnp.sort(jnp.maximum(tlo, jnp.flip(lo1, 0)))
            thi = hi2
        thr = jnp.max(jnp.where(iota == 7, tlo, _NEG))

        ngt = jnp.float32(0.0)
        for c in range(_NC):
            ngt = ngt + jnp.sum((ms[c] > thr).astype(jnp.float32))
        quota = jnp.float32(_U) - ngt

        carry = jnp.float32(0.0)
        eqoff = jnp.float32(0.0)
        for c in range(_NC):
            vc = vv[pl.ds(c * _NL, _NL)]
            pc = jnp.cumsum(vc) + carry
            carry = carry + jnp.sum(vc)
            n = (iota + (c * _NL + 1)).astype(jnp.float32)
            eqf = (ms[c] == thr).astype(jnp.float32)
            incl = jnp.cumsum(eqf)
            ex = incl - eqf + eqoff
            eqoff = eqoff + jnp.max(incl)
            sel = (ms[c] > thr) | ((eqf > 0.0) & (ex < quota))
            ov[pl.ds(c * _NL, _NL)] = jnp.where(sel, pc / n, vc)

        pltpu.sync_copy(ov, out_hbm.at[h])


def kernel(queries, keys, values):
    q = queries.reshape(_H, _L)
    k = keys.reshape(_H, _L)
    v = values.reshape(_H, _L)

    mesh = plsc.VectorSubcoreMesh(core_axis_name="c", subcore_axis_name="s",
                                  num_cores=1, num_subcores=8)
    run = pl.kernel(
        _sc_body,
        out_type=jax.ShapeDtypeStruct((_H, _L), jnp.float32),
        mesh=mesh,
        compiler_params=pltpu.CompilerParams(needs_layout_passes=False,
                                             skip_device_barrier=True),
        scratch_types=[
            pltpu.VMEM((_L,), jnp.float32),
            pltpu.VMEM((_L,), jnp.float32),
            pltpu.VMEM((_L,), jnp.float32),
            pltpu.VMEM((_L,), jnp.float32),
            pltpu.SemaphoreType.DMA,
        ],
    )
    out = run(q, k, v)
    return out.reshape(1, _H, _L, 1)
